# Initial kernel scaffold; baseline (speedup 1.0000x reference)
import jax
import jax.numpy as jnp
from jax.experimental import pallas as pl


def kernel(input_ids, wte, wpe):
    raise NotImplementedError("write your pallas kernel here")


# speedup vs baseline: 1.1609x; 1.1609x over previous
import functools

import jax
import jax.numpy as jnp
from jax import lax
from jax.experimental import pallas as pl
from jax.experimental.pallas import tpu as pltpu
from jax.experimental.pallas import tpu_sc as plsc

VOCAB = 100000
N_POS = 4096
HIDDEN = 1024
B, L = 4, 4096

NC, NS, LANES = 2, 16, 16
NW = NC * NS
TOTAL = B * L
ROWS_PER_W = TOTAL // NW
C = 32
NCHUNK = ROWS_PER_W // C


def _embed_grid():
    mesh = plsc.VectorSubcoreMesh(core_axis_name="c", subcore_axis_name="s")

    @functools.partial(
        pl.kernel,
        mesh=mesh,
        out_type=jax.ShapeDtypeStruct((TOTAL, HIDDEN), jnp.float32),
        scratch_types=[
            pltpu.VMEM((NCHUNK, C), jnp.int32),
            pltpu.VMEM((C, HIDDEN), jnp.float32),
            pltpu.VMEM((C, HIDDEN), jnp.float32),
            pltpu.SemaphoreType.DMA,
        ],
    )
    def body(ids_hbm, wte_hbm, wpe_hbm, out_hbm, idx_v, gbuf, pbuf, sem):
        wid = lax.axis_index("s") * NC + lax.axis_index("c")
        pltpu.sync_copy(ids_hbm.at[wid], idx_v)
        base_row = wid * ROWS_PER_W

        def chunk_body(cidx, carry):
            flat0 = base_row + cidx * C
            pos0 = lax.rem(flat0, L)
            pltpu.sync_copy(wpe_hbm.at[pl.ds(pos0, C)], pbuf)
            pltpu.async_copy(wte_hbm.at[idx_v.at[cidx]], gbuf, sem).wait()

            def row_body(r, c2):
                for j in range(HIDDEN // LANES):
                    sl = pl.ds(j * LANES, LANES)
                    gbuf[r, sl] = gbuf[r, sl] + pbuf[r, sl]
                return c2

            lax.fori_loop(0, C, row_body, 0)
            pltpu.sync_copy(gbuf, out_hbm.at[pl.ds(flat0, C)])
            return carry

        lax.fori_loop(0, NCHUNK, chunk_body, 0)

    return body


def kernel(input_ids, wte, wpe):
    ids = input_ids.astype(jnp.int32).reshape(NW, NCHUNK, C)
    out = _embed_grid()(ids, wte, wpe)
    return out.reshape(B, L, HIDDEN)


# speedup vs baseline: 1.9915x; 1.7155x over previous
import functools

import jax
import jax.numpy as jnp
from jax import lax
from jax.experimental import pallas as pl
from jax.experimental.pallas import tpu as pltpu
from jax.experimental.pallas import tpu_sc as plsc

VOCAB = 100000
N_POS = 4096
HIDDEN = 1024
B, L = 4, 4096

NC, NS, LANES = 2, 16, 16
NW = NC * NS
PPW = L // NW
CP = 16
NPC = PPW // CP
NB = B


def _embed():
    mesh = plsc.VectorSubcoreMesh(core_axis_name="c", subcore_axis_name="s")

    @functools.partial(
        pl.kernel,
        mesh=mesh,
        out_type=jax.ShapeDtypeStruct((B * L, HIDDEN), jnp.float32),
        scratch_types=[
            pltpu.VMEM((NPC, NB, CP), jnp.int32),
            pltpu.VMEM((CP, HIDDEN), jnp.float32),
            pltpu.VMEM((CP, HIDDEN), jnp.float32),
            pltpu.VMEM((CP, HIDDEN), jnp.float32),
            pltpu.VMEM((CP, HIDDEN), jnp.float32),
            pltpu.VMEM((CP, HIDDEN), jnp.float32),
            pltpu.SemaphoreType.DMA,
            pltpu.SemaphoreType.DMA,
            pltpu.SemaphoreType.DMA,
            pltpu.SemaphoreType.DMA,
            pltpu.SemaphoreType.DMA,
            pltpu.SemaphoreType.DMA,
            pltpu.SemaphoreType.DMA,
            pltpu.SemaphoreType.DMA,
        ],
    )
    def body(ids_hbm, wte_hbm, wpe_hbm, out_hbm, idx_v,
             g0, g1, g2, g3, pbuf, sg0, sg1, sg2, sg3, ss0, ss1, ss2, ss3):
        G = (g0, g1, g2, g3)
        SG = (sg0, sg1, sg2, sg3)
        SS = (ss0, ss1, ss2, ss3)
        wid = lax.axis_index("s") * NC + lax.axis_index("c")
        pltpu.sync_copy(ids_hbm.at[wid], idx_v)
        pos_base = wid * PPW

        def gather(pc, b):
            return pltpu.make_async_copy(wte_hbm.at[idx_v.at[pc, b]], G[b], SG[b])

        def store(pc, b):
            off = b * L + pos_base + pc * CP
            return pltpu.make_async_copy(G[b], out_hbm.at[pl.ds(off, CP)], SS[b])

        def add_pbuf(b):
            gb = G[b]

            def row(r, c):
                for j in range(HIDDEN // LANES):
                    sl = pl.ds(j * LANES, LANES)
                    gb[r, sl] = gb[r, sl] + pbuf[r, sl]
                return c

            lax.fori_loop(0, CP, row, 0)

        def load_pbuf(pc):
            pltpu.sync_copy(wpe_hbm.at[pl.ds(pos_base + pc * CP, CP)], pbuf)

        def step(pc, b, wait_prev_store, prefetch):
            if b < NB - 1:
                if wait_prev_store:
                    store(pc - 1, b + 1).wait()
                gather(pc, b + 1).start()
            elif prefetch:
                store(pc, 0).wait()
                gather(pc + 1, 0).start()
            gather(pc, b).wait()
            add_pbuf(b)
            store(pc, b).start()

        gather(0, 0).start()
        load_pbuf(0)
        for b in range(NB):
            step(0, b, wait_prev_store=False, prefetch=True)

        def group(pc, carry):
            load_pbuf(pc)
            for b in range(NB):
                step(pc, b, wait_prev_store=True, prefetch=True)
            return carry

        lax.fori_loop(1, NPC - 1, group, 0)

        load_pbuf(NPC - 1)
        for b in range(NB):
            step(NPC - 1, b, wait_prev_store=True, prefetch=False)
        for b in range(NB):
            store(NPC - 1, b).wait()

    return body


def kernel(input_ids, wte, wpe):
    ids = (input_ids.astype(jnp.int32)
           .reshape(B, NW, NPC, CP)
           .transpose(1, 2, 0, 3))
    out = _embed()(ids, wte, wpe)
    return out.reshape(B, L, HIDDEN)
